# baseline (speedup 1.0000x reference)
import functools
import math

import jax
import jax.numpy as jnp
import numpy as np
from jax import lax
from jax.experimental import pallas as pl
from jax.experimental.pallas import tpu as pltpu
from jax.experimental.pallas import tpu_sc as plsc

N = 10000
E = 320000
K1, K2, K3 = 5000, 2500, 1250

_NC, _NS = 2, 16
_NW = _NC * _NS


def _sc_gather_rows(table, idx, chunk=240):
    n, d = table.shape
    b = idx.shape[0]
    per_w = b // _NW
    assert per_w % chunk == 0 and per_w % 8 == 0
    mesh = plsc.VectorSubcoreMesh(core_axis_name="c", subcore_axis_name="s")

    @functools.partial(
        pl.kernel, mesh=mesh,
        out_type=jax.ShapeDtypeStruct((b, d), jnp.float32),
        scratch_types=[
            pltpu.VMEM((chunk,), jnp.int32),
            pltpu.VMEM((chunk, d), jnp.float32),
            pltpu.SemaphoreType.DMA,
        ],
    )
    def k(table_hbm, idx_hbm, out_hbm, idx_v, rows_v, sem):
        wid = lax.axis_index("s") * _NC + lax.axis_index("c")
        base = wid * per_w

        def body(i, carry):
            off = base + i * chunk
            pltpu.sync_copy(idx_hbm.at[pl.ds(off, chunk)], idx_v)
            pltpu.async_copy(table_hbm.at[idx_v], rows_v, sem).wait()
            pltpu.sync_copy(rows_v, out_hbm.at[pl.ds(off, chunk)])
            return carry

        lax.fori_loop(0, per_w // chunk, body, 0)

    return k(table, idx)


def _mm_body(x_ref, w_ref, o_ref):
    o_ref[...] = jnp.dot(x_ref[...], w_ref[...],
                         preferred_element_type=jnp.float32)


def _mm(x, w):
    m, k = x.shape
    k2, n = w.shape
    bm = 1000
    return pl.pallas_call(
        _mm_body,
        grid=(m // bm,),
        in_specs=[pl.BlockSpec((bm, k), lambda i: (i, 0)),
                  pl.BlockSpec((k2, n), lambda i: (0, 0))],
        out_specs=pl.BlockSpec((bm, n), lambda i: (i, 0)),
        out_shape=jax.ShapeDtypeStruct((m, n), jnp.float32),
    )(x, w)


def _rank_body(keys_ref, o_ref, *, n_pad, bi, bj):
    i = pl.program_id(0)
    ki = keys_ref[0, pl.ds(i * bi, bi)]
    ki = ki.reshape(bi, 1)
    idx_i = (jax.lax.broadcasted_iota(jnp.int32, (bi, 1), 0) + i * bi)

    def body(j, acc):
        kj = keys_ref[0, pl.ds(j * bj, bj)].reshape(1, bj)
        idx_j = jax.lax.broadcasted_iota(jnp.int32, (1, bj), 1) + j * bj
        gt = (kj > ki)
        eq = (kj == ki) & (idx_j < idx_i)
        return acc + jnp.sum((gt | eq).astype(jnp.int32), axis=1, keepdims=True)

    acc = jnp.zeros((bi, 1), jnp.int32)
    acc = jax.lax.fori_loop(0, n_pad // bj, body, acc)
    o_ref[0, pl.ds(i * bi, bi)] = acc.reshape(bi)


def _sortable(x):
    b = jax.lax.bitcast_convert_type(x, jnp.int32)
    return jnp.where(b >= 0, b, b ^ jnp.int32(0x7FFFFFFF))


def _rank(score):
    n = score.shape[0]
    n_pad = int(math.ceil(n / 512.0)) * 512
    key = _sortable(score)
    key = jnp.pad(key, (0, n_pad - n), constant_values=np.int32(-2**31))
    bi, bj = 512, 512
    rank = pl.pallas_call(
        functools.partial(_rank_body, n_pad=n_pad, bi=bi, bj=bj),
        grid=(n_pad // bi,),
        in_specs=[pl.BlockSpec((1, n_pad), lambda i: (0, 0))],
        out_specs=pl.BlockSpec((1, n_pad), lambda i: (0, 0)),
        out_shape=jax.ShapeDtypeStruct((1, n_pad), jnp.int32),
    )(key.reshape(1, n_pad))
    return rank[0, :n]


def _rank_pad(score):
    n = score.shape[0]
    n_pad = int(math.ceil(n / 512.0)) * 512
    key = _sortable(score)
    key = jnp.pad(key, (0, n_pad - n), constant_values=np.int32(-2**31))
    bi, bj = 512, 512
    rank = pl.pallas_call(
        functools.partial(_rank_body, n_pad=n_pad, bi=bi, bj=bj),
        grid=(n_pad // bi,),
        in_specs=[pl.BlockSpec((1, n_pad), lambda i: (0, 0))],
        out_specs=pl.BlockSpec((1, n_pad), lambda i: (0, 0)),
        out_shape=jax.ShapeDtypeStruct((1, n_pad), jnp.int32),
    )(key.reshape(1, n_pad))
    return rank.reshape(n_pad), jnp.pad(score, (0, n_pad - n))


def _sc_invert_gather(rank_p, score_p, table, k, pt):
    n_p = rank_p.shape[0]
    m, d = table.shape
    kp = _NW * pt
    mesh = plsc.VectorSubcoreMesh(core_axis_name="c", subcore_axis_name="s")

    @functools.partial(
        pl.kernel, mesh=mesh,
        compiler_params=pltpu.CompilerParams(needs_layout_passes=False),
        out_type=(jax.ShapeDtypeStruct((kp,), jnp.int32),
                  jax.ShapeDtypeStruct((kp,), jnp.float32),
                  jax.ShapeDtypeStruct((kp, d), jnp.float32)),
        scratch_types=[
            pltpu.VMEM((n_p,), jnp.int32),
            pltpu.VMEM((n_p,), jnp.float32),
            pltpu.VMEM((pt,), jnp.int32),
            pltpu.VMEM((pt,), jnp.float32),
            pltpu.VMEM((pt, d), jnp.float32),
            pltpu.SemaphoreType.DMA,
        ],
    )
    def kern(rank_hbm, score_hbm, table_hbm, perm_out, s_out, rows_out,
             rank_v, score_v, perm_loc, s_loc, rows_v, sem):
        wid = lax.axis_index("s") * _NC + lax.axis_index("c")
        lo = wid * pt
        hi = jnp.minimum(lo + pt, k)
        pltpu.sync_copy(rank_hbm, rank_v)
        pltpu.sync_copy(score_hbm, score_v)

        zi = jnp.zeros((16,), jnp.int32)
        zf = jnp.zeros((16,), jnp.float32)

        def initb(j, c):
            perm_loc[pl.ds(j * 16, 16)] = zi
            s_loc[pl.ds(j * 16, 16)] = zf
            return c

        lax.fori_loop(0, pt // 16, initb, 0)

        lane = jax.lax.iota(jnp.int32, 16)

        def scanb(j, c):
            rv = rank_v[pl.ds(j * 16, 16)]
            sv = score_v[pl.ds(j * 16, 16)]
            msk = (rv >= lo) & (rv < hi)
            rel = jnp.clip(rv - lo, 0, pt - 1)
            gidx = lane + j * 16
            plsc.store_scatter(perm_loc, [rel], gidx, mask=msk)
            plsc.store_scatter(s_loc, [rel], sv, mask=msk)
            return c

        lax.fori_loop(0, n_p // 16, scanb, 0)

        pltpu.sync_copy(perm_loc, perm_out.at[pl.ds(lo, pt)])
        pltpu.sync_copy(s_loc, s_out.at[pl.ds(lo, pt)])
        pltpu.async_copy(table_hbm.at[perm_loc], rows_v, sem).wait()
        pltpu.sync_copy(rows_v, rows_out.at[pl.ds(lo, pt)])

    return kern(rank_p, score_p, table)


def _sc_finish(rank3_p, score3_p, perm1_p, perm2_p, row, col):
    n3 = rank3_p.shape[0]
    np1 = perm1_p.shape[0]
    np2 = perm2_p.shape[0]
    kp3 = 1264
    per_w = E // _NW
    ch = 400
    mesh = plsc.VectorSubcoreMesh(core_axis_name="c", subcore_axis_name="s")
    i32 = jnp.int32

    @functools.partial(
        pl.kernel, mesh=mesh,
        compiler_params=pltpu.CompilerParams(needs_layout_passes=False),
        out_type=tuple([jax.ShapeDtypeStruct((kp3,), i32),
                        jax.ShapeDtypeStruct((kp3,), jnp.float32)]
                       + [jax.ShapeDtypeStruct((E,), i32)] * 6),
        scratch_types=[
            pltpu.VMEM((n3,), i32),
            pltpu.VMEM((n3,), jnp.float32),
            pltpu.VMEM((np1,), i32),
            pltpu.VMEM((np2,), i32),
            pltpu.VMEM((kp3,), i32),
            pltpu.VMEM((kp3,), jnp.float32),
            pltpu.VMEM((N,), i32),
            pltpu.VMEM((5008,), i32),
            pltpu.VMEM((2512,), i32),
            pltpu.VMEM((ch,), i32),
            pltpu.VMEM((ch,), i32),
            pltpu.VMEM((ch,), i32),
            pltpu.VMEM((ch,), i32),
            pltpu.VMEM((ch,), i32),
            pltpu.VMEM((ch,), i32),
            pltpu.VMEM((ch,), i32),
            pltpu.VMEM((ch,), i32),
        ],
    )
    def kern(rank_hbm, score_hbm, p1_hbm, p2_hbm, row_hbm, col_hbm,
             perm3_out, s3_out, o1r, o1c, o2r, o2c, o3r, o3c,
             rank_v, score_v, p1_v, p2_v, perm3_v, s3_v, m1_v, m2_v, m3_v,
             row_v, col_v, v1r, v1c, v2r, v2c, v3r, v3c):
        wid = lax.axis_index("s") * _NC + lax.axis_index("c")
        pltpu.sync_copy(rank_hbm, rank_v)
        pltpu.sync_copy(score_hbm, score_v)
        pltpu.sync_copy(p1_hbm, p1_v)
        pltpu.sync_copy(p2_hbm, p2_v)

        lane = jax.lax.iota(jnp.int32, 16)
        neg1 = jnp.full((16,), -1, i32)
        zf = jnp.zeros((16,), jnp.float32)

        def init3(j, c):
            perm3_v[pl.ds(j * 16, 16)] = jnp.zeros((16,), i32)
            s3_v[pl.ds(j * 16, 16)] = zf
            return c

        lax.fori_loop(0, kp3 // 16, init3, 0)

        def scan3(j, c):
            rv = rank_v[pl.ds(j * 16, 16)]
            sv = score_v[pl.ds(j * 16, 16)]
            msk = (rv >= 0) & (rv < K3)
            rel = jnp.clip(rv, 0, kp3 - 1)
            plsc.store_scatter(perm3_v, [rel], lane + j * 16, mask=msk)
            plsc.store_scatter(s3_v, [rel], sv, mask=msk)
            return c

        lax.fori_loop(0, n3 // 16, scan3, 0)

        @pl.when(wid == 0)
        def _():
            pltpu.sync_copy(perm3_v, perm3_out)
            pltpu.sync_copy(s3_v, s3_out)

        def initm(ref, nv):
            def b(j, c):
                ref[pl.ds(j * 16, 16)] = neg1
                return c
            lax.fori_loop(0, nv // 16, b, 0)

        initm(m1_v, N)
        initm(m2_v, 5008)
        initm(m3_v, 2512)

        def fillm(ref, src, n_src, k_real, hi):
            def b(j, c):
                gidx = lane + j * 16
                pv = src[pl.ds(j * 16, 16)]
                msk = gidx < k_real
                plsc.store_scatter(ref, [jnp.clip(pv, 0, hi)], gidx,
                                   mask=msk)
                return c
            lax.fori_loop(0, n_src // 16, b, 0)

        fillm(m1_v, p1_v, np1, K1, N - 1)
        fillm(m2_v, p2_v, np2, K2, 5007)
        fillm(m3_v, perm3_v, kp3, K3, 2511)

        base = wid * per_w

        def echunk(i, c):
            off = base + i * ch
            pltpu.sync_copy(row_hbm.at[pl.ds(off, ch)], row_v)
            pltpu.sync_copy(col_hbm.at[pl.ds(off, ch)], col_v)

            def ev(j, c2):
                sl = pl.ds(j * 16, 16)
                rr = row_v[sl]
                cc = col_v[sl]

                def step(m_ref, rr, cc, hi):
                    gr = plsc.load_gather(m_ref, [jnp.clip(rr, 0, hi)])
                    gc = plsc.load_gather(m_ref, [jnp.clip(cc, 0, hi)])
                    nr = jnp.where(rr >= 0, gr, -1)
                    nc = jnp.where(cc >= 0, gc, -1)
                    ok = (nr >= 0) & (nc >= 0)
                    return jnp.where(ok, nr, -1), jnp.where(ok, nc, -1)

                a1, b1_ = step(m1_v, rr, cc, N - 1)
                a2, b2_ = step(m2_v, a1, b1_, K1 - 1)
                a3, b3_ = step(m3_v, a2, b2_, K2 - 1)
                v1r[sl] = a1
                v1c[sl] = b1_
                v2r[sl] = a2
                v2c[sl] = b2_
                v3r[sl] = a3
                v3c[sl] = b3_
                return c2

            lax.fori_loop(0, ch // 16, ev, 0)
            pltpu.sync_copy(v1r, o1r.at[pl.ds(off, ch)])
            pltpu.sync_copy(v1c, o1c.at[pl.ds(off, ch)])
            pltpu.sync_copy(v2r, o2r.at[pl.ds(off, ch)])
            pltpu.sync_copy(v2c, o2c.at[pl.ds(off, ch)])
            pltpu.sync_copy(v3r, o3r.at[pl.ds(off, ch)])
            pltpu.sync_copy(v3c, o3c.at[pl.ds(off, ch)])
            return c

        lax.fori_loop(0, per_w // ch, echunk, 0)

    return kern(rank3_p, score3_p, perm1_p, perm2_p, row, col)


def kernel(x, edge_index, W1, b1, W2, b2, W3, b3, p1, p2, p3):
    row, col = edge_index[0], edge_index[1]
    loops = jnp.arange(N, dtype=row.dtype)
    r_all = jnp.concatenate([row, loops])
    c_all = jnp.concatenate([col, loops])
    deg = jax.ops.segment_sum(jnp.ones(r_all.shape[0], jnp.float32), r_all,
                              num_segments=N)
    dinv = jnp.where(deg > 0, 1.0 / jnp.sqrt(deg), 0.0)
    norm = dinv[r_all] * dinv[c_all]

    n_upd = E + N
    n_pad = ((n_upd + _NW * 240 - 1) // (_NW * 240)) * (_NW * 240)
    c_pad = jnp.concatenate([c_all, jnp.zeros(n_pad - n_upd, jnp.int32)])

    def conv(h, W, b):
        hw = _mm(h, W)
        upd = _sc_gather_rows(hw, c_pad)[:n_upd] * norm[:, None]
        return jax.ops.segment_sum(upd, r_all, num_segments=N) + b

    h = jax.nn.relu(conv(x, W1, b1))
    h = jax.nn.relu(conv(h, W2, b2))
    h = jax.nn.relu(conv(h, W3, b3))

    score1 = (h @ p1) / (jnp.linalg.norm(p1) + 1e-12)
    rank1p, score1p = _rank_pad(score1)
    perm1p, s1p, x1raw = _sc_invert_gather(rank1p, score1p, h, K1, pt=160)
    perm1, s1 = perm1p[:K1], s1p[:K1]
    x1 = x1raw[:K1] * jnp.tanh(s1)[:, None]
    l1 = jnp.mean(1.0 - jnp.tanh(s1))
    score2 = (x1 @ p2) / (jnp.linalg.norm(p2) + 1e-12)
    rank2p, score2p = _rank_pad(score2)
    perm2p, s2p, x2raw = _sc_invert_gather(rank2p, score2p, x1, K2, pt=80)
    perm2, s2 = perm2p[:K2], s2p[:K2]
    x2 = x2raw[:K2] * jnp.tanh(s2)[:, None]
    l2 = jnp.mean(1.0 - jnp.tanh(s2))
    score3 = (x2 @ p3) / (jnp.linalg.norm(p3) + 1e-12)
    rank3p, score3p = _rank_pad(score3)
    (perm3p, s3p, a1, c1_, a2, c2_, a3, c3_) = _sc_finish(
        rank3p, score3p, perm1p, perm2p, row, col)
    perm3, s3 = perm3p[:K3], s3p[:K3]
    l3 = jnp.mean(1.0 - jnp.tanh(s3))

    ei1 = jnp.stack([a1, c1_])
    ei2 = jnp.stack([a2, c2_])
    ei3 = jnp.stack([a3, c3_])
    return (ei1, s1, perm1, ei2, s2, perm2, ei3, s3, perm3, l1 + l2 + l3)


# speedup vs baseline: 1.4103x; 1.4103x over previous
import functools
import math

import jax
import jax.numpy as jnp
import numpy as np
from jax import lax
from jax.experimental import pallas as pl
from jax.experimental.pallas import tpu as pltpu
from jax.experimental.pallas import tpu_sc as plsc

N = 10000
E = 320000
K1, K2, K3 = 5000, 2500, 1250

_NC, _NS = 2, 16
_NW = _NC * _NS


def _sc_gather_rows(table, idx, chunk=240):
    n, d = table.shape
    b = idx.shape[0]
    per_w = b // _NW
    assert per_w % chunk == 0 and per_w % 8 == 0
    mesh = plsc.VectorSubcoreMesh(core_axis_name="c", subcore_axis_name="s")

    @functools.partial(
        pl.kernel, mesh=mesh,
        out_type=jax.ShapeDtypeStruct((b, d), jnp.float32),
        scratch_types=[
            pltpu.VMEM((chunk,), jnp.int32),
            pltpu.VMEM((chunk, d), jnp.float32),
            pltpu.SemaphoreType.DMA,
        ],
    )
    def k(table_hbm, idx_hbm, out_hbm, idx_v, rows_v, sem):
        wid = lax.axis_index("s") * _NC + lax.axis_index("c")
        base = wid * per_w

        def body(i, carry):
            off = base + i * chunk
            pltpu.sync_copy(idx_hbm.at[pl.ds(off, chunk)], idx_v)
            pltpu.async_copy(table_hbm.at[idx_v], rows_v, sem).wait()
            pltpu.sync_copy(rows_v, out_hbm.at[pl.ds(off, chunk)])
            return carry

        lax.fori_loop(0, per_w // chunk, body, 0)

    return k(table, idx)


_CAP = 16384
_RPT = 320


def _sc_compact_edges(r_pad, c_pad, norm_pad):
    b = r_pad.shape[0]
    ch = 2560
    assert b % ch == 0
    mesh = plsc.VectorSubcoreMesh(core_axis_name="c", subcore_axis_name="s")
    i32 = jnp.int32

    @functools.partial(
        pl.kernel, mesh=mesh,
        compiler_params=pltpu.CompilerParams(needs_layout_passes=False),
        out_type=(jax.ShapeDtypeStruct((_NW, _CAP), i32),
                  jax.ShapeDtypeStruct((_NW, _CAP), jnp.float32),
                  jax.ShapeDtypeStruct((_NW, _CAP), i32),
                  jax.ShapeDtypeStruct((_NW, 16), i32)),
        scratch_types=[
            pltpu.VMEM((ch,), i32),
            pltpu.VMEM((ch,), i32),
            pltpu.VMEM((ch,), jnp.float32),
            pltpu.VMEM((_CAP,), i32),
            pltpu.VMEM((_CAP,), jnp.float32),
            pltpu.VMEM((_CAP,), i32),
            pltpu.VMEM((16,), i32),
        ],
    )
    def kern(r_hbm, c_hbm, n_hbm, cl_out, nl_out, rl_out, cnt_out,
             r_v, c_v, n_v, cl_v, nl_v, rl_v, cnt_v):
        wid = lax.axis_index("s") * _NC + lax.axis_index("c")
        lo = wid * _RPT
        hi = jnp.minimum(lo + _RPT, N)

        zc = jnp.zeros((16,), i32)
        zn = jnp.zeros((16,), jnp.float32)

        def prefill(j, c):
            cl_v[pl.ds(j * 16, 16)] = zc
            nl_v[pl.ds(j * 16, 16)] = zn
            rl_v[pl.ds(j * 16, 16)] = jnp.full((16,), lo, i32)
            return c

        lax.fori_loop(0, _CAP // 16, prefill, 0)

        def chunk(g, cnt):
            off = g * ch
            pltpu.sync_copy(r_hbm.at[pl.ds(off, ch)], r_v)
            pltpu.sync_copy(c_hbm.at[pl.ds(off, ch)], c_v)
            pltpu.sync_copy(n_hbm.at[pl.ds(off, ch)], n_v)

            def vec(j, cnt):
                sl = pl.ds(j * 16, 16)
                rv = r_v[sl]
                msk = (rv >= lo) & (rv < hi)
                pos = jnp.minimum(cnt, _CAP - 16)
                plsc.store_compressed(cl_v.at[pl.ds(pos, 16)], c_v[sl], mask=msk)
                plsc.store_compressed(nl_v.at[pl.ds(pos, 16)], n_v[sl], mask=msk)
                plsc.store_compressed(rl_v.at[pl.ds(pos, 16)], rv, mask=msk)
                return cnt + jnp.sum(msk.astype(i32))

            return lax.fori_loop(0, ch // 16, vec, cnt)

        cnt = lax.fori_loop(0, b // ch, chunk, jnp.int32(0))

        pltpu.sync_copy(cl_v, cl_out.at[wid])
        pltpu.sync_copy(nl_v, nl_out.at[wid])
        pltpu.sync_copy(rl_v, rl_out.at[wid])
        cnt_v[pl.ds(0, 16)] = jnp.full((16,), cnt, i32)
        pltpu.sync_copy(cnt_v, cnt_out.at[wid])

    return kern(r_pad, c_pad, norm_pad)


def _sc_aggregate(table, cl, nl, rl, cnt):
    n, d = table.shape
    g = 64
    nv = d // 16
    mesh = plsc.VectorSubcoreMesh(core_axis_name="c", subcore_axis_name="s")
    i32 = jnp.int32

    @functools.partial(
        pl.kernel, mesh=mesh,
        compiler_params=pltpu.CompilerParams(needs_layout_passes=False),
        out_type=jax.ShapeDtypeStruct((_NW * _RPT, d), jnp.float32),
        scratch_types=[
            pltpu.VMEM((_RPT, d), jnp.float32),
            pltpu.VMEM((g,), i32),
            pltpu.VMEM((g,), jnp.float32),
            pltpu.VMEM((g,), i32),
            pltpu.VMEM((g, d), jnp.float32),
            pltpu.VMEM((16,), i32),
            pltpu.SemaphoreType.DMA,
        ],
    )
    def kern(table_hbm, cl_hbm, nl_hbm, rl_hbm, cnt_hbm, out_hbm,
             acc_v, c_v, n_v, r_v, rows_v, cnt_v, sem):
        wid = lax.axis_index("s") * _NC + lax.axis_index("c")
        lo = wid * _RPT

        zf = jnp.zeros((16,), jnp.float32)

        def zrow(j, c):
            r = j // nv
            f = j % nv
            acc_v[r, pl.ds(f * 16, 16)] = zf
            return c

        lax.fori_loop(0, _RPT * nv, zrow, 0)

        pltpu.sync_copy(cnt_hbm.at[wid], cnt_v)
        cnt = cnt_v[pl.ds(0, 16)][0]
        nch = (cnt + g - 1) // g

        def chunk(gi, c):
            off = gi * g
            pltpu.sync_copy(cl_hbm.at[wid, pl.ds(off, g)], c_v)
            pltpu.sync_copy(nl_hbm.at[wid, pl.ds(off, g)], n_v)
            pltpu.sync_copy(rl_hbm.at[wid, pl.ds(off, g)], r_v)
            pltpu.async_copy(table_hbm.at[c_v], rows_v, sem).wait()

            def grp(q, c2):
                rv = r_v[pl.ds(q * 16, 16)]
                nvv = n_v[pl.ds(q * 16, 16)]
                rel = rv - lo
                for l in range(16):
                    e = q * 16 + l
                    nrm = jnp.full((16,), nvv[l])
                    rr = rel[l]
                    for f in range(nv):
                        term = rows_v[e, pl.ds(f * 16, 16)] * nrm
                        plsc.addupdate(acc_v.at[rr, pl.ds(f * 16, 16)],
                                       term)
                return c2

            lax.fori_loop(0, g // 16, grp, 0)
            return c

        lax.fori_loop(0, nch, chunk, 0)
        pltpu.sync_copy(acc_v, out_hbm.at[pl.ds(lo, _RPT)])

    return kern(table, cl, nl, rl, cnt)


def _mm_body(x_ref, w_ref, o_ref):
    o_ref[...] = jnp.dot(x_ref[...], w_ref[...],
                         preferred_element_type=jnp.float32)


def _mm(x, w):
    m, k = x.shape
    k2, n = w.shape
    bm = 1000
    return pl.pallas_call(
        _mm_body,
        grid=(m // bm,),
        in_specs=[pl.BlockSpec((bm, k), lambda i: (i, 0)),
                  pl.BlockSpec((k2, n), lambda i: (0, 0))],
        out_specs=pl.BlockSpec((bm, n), lambda i: (i, 0)),
        out_shape=jax.ShapeDtypeStruct((m, n), jnp.float32),
    )(x, w)


def _rank_body(keys_ref, o_ref, *, n_pad, bi, bj):
    i = pl.program_id(0)
    ki = keys_ref[0, pl.ds(i * bi, bi)]
    ki = ki.reshape(bi, 1)
    idx_i = (jax.lax.broadcasted_iota(jnp.int32, (bi, 1), 0) + i * bi)

    def body(j, acc):
        kj = keys_ref[0, pl.ds(j * bj, bj)].reshape(1, bj)
        idx_j = jax.lax.broadcasted_iota(jnp.int32, (1, bj), 1) + j * bj
        gt = (kj > ki)
        eq = (kj == ki) & (idx_j < idx_i)
        return acc + jnp.sum((gt | eq).astype(jnp.int32), axis=1, keepdims=True)

    acc = jnp.zeros((bi, 1), jnp.int32)
    acc = jax.lax.fori_loop(0, n_pad // bj, body, acc)
    o_ref[0, pl.ds(i * bi, bi)] = acc.reshape(bi)


def _sortable(x):
    b = jax.lax.bitcast_convert_type(x, jnp.int32)
    return jnp.where(b >= 0, b, b ^ jnp.int32(0x7FFFFFFF))


def _rank(score):
    n = score.shape[0]
    n_pad = int(math.ceil(n / 512.0)) * 512
    key = _sortable(score)
    key = jnp.pad(key, (0, n_pad - n), constant_values=np.int32(-2**31))
    bi, bj = 512, 512
    rank = pl.pallas_call(
        functools.partial(_rank_body, n_pad=n_pad, bi=bi, bj=bj),
        grid=(n_pad // bi,),
        in_specs=[pl.BlockSpec((1, n_pad), lambda i: (0, 0))],
        out_specs=pl.BlockSpec((1, n_pad), lambda i: (0, 0)),
        out_shape=jax.ShapeDtypeStruct((1, n_pad), jnp.int32),
    )(key.reshape(1, n_pad))
    return rank[0, :n]


def _rank_pad(score):
    n = score.shape[0]
    n_pad = int(math.ceil(n / 512.0)) * 512
    key = _sortable(score)
    key = jnp.pad(key, (0, n_pad - n), constant_values=np.int32(-2**31))
    bi, bj = 512, 512
    rank = pl.pallas_call(
        functools.partial(_rank_body, n_pad=n_pad, bi=bi, bj=bj),
        grid=(n_pad // bi,),
        in_specs=[pl.BlockSpec((1, n_pad), lambda i: (0, 0))],
        out_specs=pl.BlockSpec((1, n_pad), lambda i: (0, 0)),
        out_shape=jax.ShapeDtypeStruct((1, n_pad), jnp.int32),
    )(key.reshape(1, n_pad))
    return rank.reshape(n_pad), jnp.pad(score, (0, n_pad - n))


def _sc_invert_gather(rank_p, score_p, table, k, pt):
    n_p = rank_p.shape[0]
    m, d = table.shape
    kp = _NW * pt
    mesh = plsc.VectorSubcoreMesh(core_axis_name="c", subcore_axis_name="s")

    @functools.partial(
        pl.kernel, mesh=mesh,
        compiler_params=pltpu.CompilerParams(needs_layout_passes=False),
        out_type=(jax.ShapeDtypeStruct((kp,), jnp.int32),
                  jax.ShapeDtypeStruct((kp,), jnp.float32),
                  jax.ShapeDtypeStruct((kp, d), jnp.float32)),
        scratch_types=[
            pltpu.VMEM((n_p,), jnp.int32),
            pltpu.VMEM((n_p,), jnp.float32),
            pltpu.VMEM((pt,), jnp.int32),
            pltpu.VMEM((pt,), jnp.float32),
            pltpu.VMEM((pt, d), jnp.float32),
            pltpu.SemaphoreType.DMA,
        ],
    )
    def kern(rank_hbm, score_hbm, table_hbm, perm_out, s_out, rows_out,
             rank_v, score_v, perm_loc, s_loc, rows_v, sem):
        wid = lax.axis_index("s") * _NC + lax.axis_index("c")
        lo = wid * pt
        hi = jnp.minimum(lo + pt, k)
        pltpu.sync_copy(rank_hbm, rank_v)
        pltpu.sync_copy(score_hbm, score_v)

        zi = jnp.zeros((16,), jnp.int32)
        zf = jnp.zeros((16,), jnp.float32)

        def initb(j, c):
            perm_loc[pl.ds(j * 16, 16)] = zi
            s_loc[pl.ds(j * 16, 16)] = zf
            return c

        lax.fori_loop(0, pt // 16, initb, 0)

        lane = jax.lax.iota(jnp.int32, 16)

        def scanb(j, c):
            rv = rank_v[pl.ds(j * 16, 16)]
            sv = score_v[pl.ds(j * 16, 16)]
            msk = (rv >= lo) & (rv < hi)
            rel = jnp.clip(rv - lo, 0, pt - 1)
            gidx = lane + j * 16
            plsc.store_scatter(perm_loc, [rel], gidx, mask=msk)
            plsc.store_scatter(s_loc, [rel], sv, mask=msk)
            return c

        lax.fori_loop(0, n_p // 16, scanb, 0)

        pltpu.sync_copy(perm_loc, perm_out.at[pl.ds(lo, pt)])
        pltpu.sync_copy(s_loc, s_out.at[pl.ds(lo, pt)])
        pltpu.async_copy(table_hbm.at[perm_loc], rows_v, sem).wait()
        pltpu.sync_copy(rows_v, rows_out.at[pl.ds(lo, pt)])

    return kern(rank_p, score_p, table)


def _sc_finish(rank3_p, score3_p, perm1_p, perm2_p, row, col):
    n3 = rank3_p.shape[0]
    np1 = perm1_p.shape[0]
    np2 = perm2_p.shape[0]
    kp3 = 1264
    per_w = E // _NW
    ch = 400
    mesh = plsc.VectorSubcoreMesh(core_axis_name="c", subcore_axis_name="s")
    i32 = jnp.int32

    @functools.partial(
        pl.kernel, mesh=mesh,
        compiler_params=pltpu.CompilerParams(needs_layout_passes=False),
        out_type=tuple([jax.ShapeDtypeStruct((kp3,), i32),
                        jax.ShapeDtypeStruct((kp3,), jnp.float32)]
                       + [jax.ShapeDtypeStruct((E,), i32)] * 6),
        scratch_types=[
            pltpu.VMEM((n3,), i32),
            pltpu.VMEM((n3,), jnp.float32),
            pltpu.VMEM((np1,), i32),
            pltpu.VMEM((np2,), i32),
            pltpu.VMEM((kp3,), i32),
            pltpu.VMEM((kp3,), jnp.float32),
            pltpu.VMEM((N,), i32),
            pltpu.VMEM((5008,), i32),
            pltpu.VMEM((2512,), i32),
            pltpu.VMEM((ch,), i32),
            pltpu.VMEM((ch,), i32),
            pltpu.VMEM((ch,), i32),
            pltpu.VMEM((ch,), i32),
            pltpu.VMEM((ch,), i32),
            pltpu.VMEM((ch,), i32),
            pltpu.VMEM((ch,), i32),
            pltpu.VMEM((ch,), i32),
        ],
    )
    def kern(rank_hbm, score_hbm, p1_hbm, p2_hbm, row_hbm, col_hbm,
             perm3_out, s3_out, o1r, o1c, o2r, o2c, o3r, o3c,
             rank_v, score_v, p1_v, p2_v, perm3_v, s3_v, m1_v, m2_v, m3_v,
             row_v, col_v, v1r, v1c, v2r, v2c, v3r, v3c):
        wid = lax.axis_index("s") * _NC + lax.axis_index("c")
        pltpu.sync_copy(rank_hbm, rank_v)
        pltpu.sync_copy(score_hbm, score_v)
        pltpu.sync_copy(p1_hbm, p1_v)
        pltpu.sync_copy(p2_hbm, p2_v)

        lane = jax.lax.iota(jnp.int32, 16)
        neg1 = jnp.full((16,), -1, i32)
        zf = jnp.zeros((16,), jnp.float32)

        def init3(j, c):
            perm3_v[pl.ds(j * 16, 16)] = jnp.zeros((16,), i32)
            s3_v[pl.ds(j * 16, 16)] = zf
            return c

        lax.fori_loop(0, kp3 // 16, init3, 0)

        def scan3(j, c):
            rv = rank_v[pl.ds(j * 16, 16)]
            sv = score_v[pl.ds(j * 16, 16)]
            msk = (rv >= 0) & (rv < K3)
            rel = jnp.clip(rv, 0, kp3 - 1)
            plsc.store_scatter(perm3_v, [rel], lane + j * 16, mask=msk)
            plsc.store_scatter(s3_v, [rel], sv, mask=msk)
            return c

        lax.fori_loop(0, n3 // 16, scan3, 0)

        @pl.when(wid == 0)
        def _():
            pltpu.sync_copy(perm3_v, perm3_out)
            pltpu.sync_copy(s3_v, s3_out)

        def initm(ref, nv):
            def b(j, c):
                ref[pl.ds(j * 16, 16)] = neg1
                return c
            lax.fori_loop(0, nv // 16, b, 0)

        initm(m1_v, N)
        initm(m2_v, 5008)
        initm(m3_v, 2512)

        def fillm(ref, src, n_src, k_real, hi):
            def b(j, c):
                gidx = lane + j * 16
                pv = src[pl.ds(j * 16, 16)]
                msk = gidx < k_real
                plsc.store_scatter(ref, [jnp.clip(pv, 0, hi)], gidx,
                                   mask=msk)
                return c
            lax.fori_loop(0, n_src // 16, b, 0)

        fillm(m1_v, p1_v, np1, K1, N - 1)
        fillm(m2_v, p2_v, np2, K2, 5007)
        fillm(m3_v, perm3_v, kp3, K3, 2511)

        base = wid * per_w

        def echunk(i, c):
            off = base + i * ch
            pltpu.sync_copy(row_hbm.at[pl.ds(off, ch)], row_v)
            pltpu.sync_copy(col_hbm.at[pl.ds(off, ch)], col_v)

            def ev(j, c2):
                sl = pl.ds(j * 16, 16)
                rr = row_v[sl]
                cc = col_v[sl]

                def step(m_ref, rr, cc, hi):
                    gr = plsc.load_gather(m_ref, [jnp.clip(rr, 0, hi)])
                    gc = plsc.load_gather(m_ref, [jnp.clip(cc, 0, hi)])
                    nr = jnp.where(rr >= 0, gr, -1)
                    nc = jnp.where(cc >= 0, gc, -1)
                    ok = (nr >= 0) & (nc >= 0)
                    return jnp.where(ok, nr, -1), jnp.where(ok, nc, -1)

                a1, b1_ = step(m1_v, rr, cc, N - 1)
                a2, b2_ = step(m2_v, a1, b1_, K1 - 1)
                a3, b3_ = step(m3_v, a2, b2_, K2 - 1)
                v1r[sl] = a1
                v1c[sl] = b1_
                v2r[sl] = a2
                v2c[sl] = b2_
                v3r[sl] = a3
                v3c[sl] = b3_
                return c2

            lax.fori_loop(0, ch // 16, ev, 0)
            pltpu.sync_copy(v1r, o1r.at[pl.ds(off, ch)])
            pltpu.sync_copy(v1c, o1c.at[pl.ds(off, ch)])
            pltpu.sync_copy(v2r, o2r.at[pl.ds(off, ch)])
            pltpu.sync_copy(v2c, o2c.at[pl.ds(off, ch)])
            pltpu.sync_copy(v3r, o3r.at[pl.ds(off, ch)])
            pltpu.sync_copy(v3c, o3c.at[pl.ds(off, ch)])
            return c

        lax.fori_loop(0, per_w // ch, echunk, 0)

    return kern(rank3_p, score3_p, perm1_p, perm2_p, row, col)


def kernel(x, edge_index, W1, b1, W2, b2, W3, b3, p1, p2, p3):
    row, col = edge_index[0], edge_index[1]
    loops = jnp.arange(N, dtype=row.dtype)
    r_all = jnp.concatenate([row, loops])
    c_all = jnp.concatenate([col, loops])
    deg = jax.ops.segment_sum(jnp.ones(r_all.shape[0], jnp.float32), r_all,
                              num_segments=N)
    dinv = jnp.where(deg > 0, 1.0 / jnp.sqrt(deg), 0.0)
    norm = dinv[r_all] * dinv[c_all]

    n_upd = E + N
    n_pad = ((n_upd + 2560 - 1) // 2560) * 2560
    pad = n_pad - n_upd
    c_pad = jnp.concatenate([c_all, jnp.zeros(pad, jnp.int32)])
    r_pad = jnp.concatenate([r_all, jnp.full(pad, 2**30, jnp.int32)])
    norm_pad = jnp.concatenate([norm, jnp.zeros(pad, jnp.float32)])

    cl, nl, rl, cnt = _sc_compact_edges(r_pad, c_pad, norm_pad)

    def conv(h, W, b):
        hw = _mm(h, W)
        return _sc_aggregate(hw, cl, nl, rl, cnt)[:N] + b

    h = jax.nn.relu(conv(x, W1, b1))
    h = jax.nn.relu(conv(h, W2, b2))
    h = jax.nn.relu(conv(h, W3, b3))

    score1 = (h @ p1) / (jnp.linalg.norm(p1) + 1e-12)
    rank1p, score1p = _rank_pad(score1)
    perm1p, s1p, x1raw = _sc_invert_gather(rank1p, score1p, h, K1, pt=160)
    perm1, s1 = perm1p[:K1], s1p[:K1]
    x1 = x1raw[:K1] * jnp.tanh(s1)[:, None]
    l1 = jnp.mean(1.0 - jnp.tanh(s1))
    score2 = (x1 @ p2) / (jnp.linalg.norm(p2) + 1e-12)
    rank2p, score2p = _rank_pad(score2)
    perm2p, s2p, x2raw = _sc_invert_gather(rank2p, score2p, x1, K2, pt=80)
    perm2, s2 = perm2p[:K2], s2p[:K2]
    x2 = x2raw[:K2] * jnp.tanh(s2)[:, None]
    l2 = jnp.mean(1.0 - jnp.tanh(s2))
    score3 = (x2 @ p3) / (jnp.linalg.norm(p3) + 1e-12)
    rank3p, score3p = _rank_pad(score3)
    (perm3p, s3p, a1, c1_, a2, c2_, a3, c3_) = _sc_finish(
        rank3p, score3p, perm1p, perm2p, row, col)
    perm3, s3 = perm3p[:K3], s3p[:K3]
    l3 = jnp.mean(1.0 - jnp.tanh(s3))

    ei1 = jnp.stack([a1, c1_])
    ei2 = jnp.stack([a2, c2_])
    ei3 = jnp.stack([a3, c3_])
    return (ei1, s1, perm1, ei2, s2, perm2, ei3, s3, perm3, l1 + l2 + l3)
